# Initial kernel scaffold; baseline (speedup 1.0000x reference)
import jax
import jax.numpy as jnp
from jax.experimental import pallas as pl


def kernel(tokens, wte, wpe):
    raise NotImplementedError("write your pallas kernel here")


# speedup vs baseline: 1.0250x; 1.0250x over previous
import functools

import jax
import jax.numpy as jnp
from jax import lax
from jax.experimental import pallas as pl
from jax.experimental.pallas import tpu as pltpu
from jax.experimental.pallas import tpu_sc as plsc

BATCH = 4
SEQ = 2048
D_MODEL = 1024

_info = plsc.get_sparse_core_info()
NC, NS = _info.num_cores, _info.num_subcores
NW = NC * NS
POS_PER_W = SEQ // NW
CHUNK = 32


def _emb_kernel(tok_hbm, wte_hbm, wpe_hbm, out_hbm, idx_v, wpe_v, acc_v, sem):
    wid = lax.axis_index("s") * NC + lax.axis_index("c")
    pos0 = wid * POS_PER_W
    for pc in range(POS_PER_W // CHUNK):
        p = pos0 + pc * CHUNK
        pltpu.sync_copy(wpe_hbm.at[pl.ds(p, CHUNK)], wpe_v)
        for b in range(BATCH):
            base = b * SEQ + p
            pltpu.sync_copy(tok_hbm.at[pl.ds(base, CHUNK)], idx_v)
            pltpu.async_copy(wte_hbm.at[idx_v], acc_v, sem).wait()

            def row_body(r):
                for c in range(D_MODEL // 16):
                    x = wpe_v[r, pl.ds(c * 16, 16)]
                    plsc.addupdate(acc_v.at[r, pl.ds(c * 16, 16)], x)

            plsc.parallel_loop(0, CHUNK)(row_body)
            pltpu.sync_copy(acc_v, out_hbm.at[pl.ds(base, CHUNK)])


@jax.jit
def _run(tok_flat, wte, wpe):
    mesh = plsc.VectorSubcoreMesh(core_axis_name="c", subcore_axis_name="s")
    f = pl.kernel(
        _emb_kernel,
        out_type=jax.ShapeDtypeStruct((BATCH * SEQ, D_MODEL), jnp.float32),
        mesh=mesh,
        scratch_types=[
            pltpu.VMEM((CHUNK,), jnp.int32),
            pltpu.VMEM((CHUNK, D_MODEL), jnp.float32),
            pltpu.VMEM((CHUNK, D_MODEL), jnp.float32),
            pltpu.SemaphoreType.DMA,
        ],
    )
    return f(tok_flat, wte, wpe)


def kernel(tokens, wte, wpe):
    tok_flat = tokens.reshape(-1).astype(jnp.int32)
    out = _run(tok_flat, wte, wpe)
    return out.reshape(BATCH, SEQ, D_MODEL)


# speedup vs baseline: 1.2770x; 1.2458x over previous
import jax
import jax.numpy as jnp
from jax import lax
from jax.experimental import pallas as pl
from jax.experimental.pallas import tpu as pltpu
from jax.experimental.pallas import tpu_sc as plsc

BATCH = 4
SEQ = 2048
D_MODEL = 1024

_info = plsc.get_sparse_core_info()
NC, NS = _info.num_cores, _info.num_subcores
NW = NC * NS
POS_PER_W = SEQ // NW
CHUNK = 16
NPC = POS_PER_W // CHUNK
NROUND = NPC * BATCH


def _emb_kernel(tok_hbm, wte_hbm, wpe_hbm, out_hbm,
                idx_v, wpe0, wpe1, acc0, acc1,
                gsem0, gsem1, ssem0, ssem1, wsem0, wsem1, isem):
    wid = lax.axis_index("s") * NC + lax.axis_index("c")
    pos0 = wid * POS_PER_W
    acc = (acc0, acc1)
    wpe = (wpe0, wpe1)
    gsem = (gsem0, gsem1)
    ssem = (ssem0, ssem1)
    wsem = (wsem0, wsem1)

    idx_descs = [
        pltpu.async_copy(tok_hbm.at[pl.ds(b * SEQ + pos0, POS_PER_W)],
                         idx_v.at[b], isem)
        for b in range(BATCH)
    ]
    wpe_descs = {}
    for pc in range(2):
        wpe_descs[pc] = pltpu.async_copy(
            wpe_hbm.at[pl.ds(pos0 + pc * CHUNK, CHUNK)], wpe[pc], wsem[pc])
    for d in idx_descs:
        d.wait()

    def gather(r):
        pc, b = divmod(r, BATCH)
        return pltpu.async_copy(
            wte_hbm.at[idx_v.at[b, pl.ds(pc * CHUNK, CHUNK)]],
            acc[r % 2], gsem[r % 2])

    g_descs = {0: gather(0)}
    s_descs = {}
    for r in range(NROUND):
        buf = r % 2
        pc, b = divmod(r, BATCH)
        if r + 1 < NROUND:
            if r - 1 in s_descs:
                s_descs[r - 1].wait()
            g_descs[r + 1] = gather(r + 1)
        g_descs[r].wait()
        if b == 0:
            wpe_descs[pc].wait()
        a, w = acc[buf], wpe[pc % 2]

        def row_body(row):
            for c in range(D_MODEL // 16):
                x = w[row, pl.ds(c * 16, 16)]
                plsc.addupdate(a.at[row, pl.ds(c * 16, 16)], x)

        plsc.parallel_loop(0, CHUNK)(row_body)
        s_descs[r] = pltpu.async_copy(
            a, out_hbm.at[pl.ds(b * SEQ + pos0 + pc * CHUNK, CHUNK)],
            ssem[buf])
        if b == BATCH - 1 and pc + 2 < NPC:
            wpe_descs[pc + 2] = pltpu.async_copy(
                wpe_hbm.at[pl.ds(pos0 + (pc + 2) * CHUNK, CHUNK)],
                wpe[pc % 2], wsem[pc % 2])
    s_descs[NROUND - 2].wait()
    s_descs[NROUND - 1].wait()


@jax.jit
def _run(tok_flat, wte, wpe):
    mesh = plsc.VectorSubcoreMesh(core_axis_name="c", subcore_axis_name="s")
    f = pl.kernel(
        _emb_kernel,
        out_type=jax.ShapeDtypeStruct((BATCH * SEQ, D_MODEL), jnp.float32),
        mesh=mesh,
        scratch_types=[
            pltpu.VMEM((BATCH, POS_PER_W), jnp.int32),
            pltpu.VMEM((CHUNK, D_MODEL), jnp.float32),
            pltpu.VMEM((CHUNK, D_MODEL), jnp.float32),
            pltpu.VMEM((CHUNK, D_MODEL), jnp.float32),
            pltpu.VMEM((CHUNK, D_MODEL), jnp.float32),
            pltpu.SemaphoreType.DMA,
            pltpu.SemaphoreType.DMA,
            pltpu.SemaphoreType.DMA,
            pltpu.SemaphoreType.DMA,
            pltpu.SemaphoreType.DMA,
            pltpu.SemaphoreType.DMA,
            pltpu.SemaphoreType.DMA,
        ],
    )
    return f(tok_flat, wte, wpe)


def kernel(tokens, wte, wpe):
    tok_flat = tokens.reshape(-1).astype(jnp.int32)
    out = _run(tok_flat, wte, wpe)
    return out.reshape(BATCH, SEQ, D_MODEL)
